# Initial kernel scaffold; baseline (speedup 1.0000x reference)
import jax
import jax.numpy as jnp
from jax.experimental import pallas as pl


def kernel(pointcloud, params):
    raise NotImplementedError("write your pallas kernel here")


# speedup vs baseline: 1.2137x; 1.2137x over previous
import functools

import jax
import jax.numpy as jnp
from jax.experimental import pallas as pl


def _fps_body(px_ref, py_ref, pz_ref, sx_ref, sy_ref, sz_ref, *, S, N):
    px = px_ref[...]
    py = py_ref[...]
    pz = pz_ref[...]
    B = px.shape[0]
    iota = jax.lax.broadcasted_iota(jnp.int32, (B, N), 1)
    cols = jax.lax.broadcasted_iota(jnp.int32, (B, S), 1)

    def body(s, carry):
        dist, far = carry
        onehot = iota == far
        cx = jnp.sum(jnp.where(onehot, px, 0.0), axis=1, keepdims=True)
        cy = jnp.sum(jnp.where(onehot, py, 0.0), axis=1, keepdims=True)
        cz = jnp.sum(jnp.where(onehot, pz, 0.0), axis=1, keepdims=True)
        hit = cols == s
        sx_ref[...] = jnp.where(hit, cx, sx_ref[...])
        sy_ref[...] = jnp.where(hit, cy, sy_ref[...])
        sz_ref[...] = jnp.where(hit, cz, sz_ref[...])
        dx = px - cx
        dy = py - cy
        dz = pz - cz
        d = (dx * dx + dy * dy) + dz * dz
        dist = jnp.minimum(dist, d)
        dmax = jnp.max(dist, axis=1, keepdims=True)
        far = jnp.min(jnp.where(dist == dmax, iota, N), axis=1, keepdims=True)
        return dist, far

    init = (jnp.full((B, N), 1e10, dtype=jnp.float32),
            jnp.zeros((B, 1), dtype=jnp.int32))
    jax.lax.fori_loop(0, S, body, init)


def _fps_new_xyz(xyz, S, interpret=False):
    B, N, _ = xyz.shape
    px, py, pz = xyz[..., 0], xyz[..., 1], xyz[..., 2]
    out = pl.pallas_call(
        functools.partial(_fps_body, S=S, N=N),
        out_shape=[jax.ShapeDtypeStruct((B, S), jnp.float32)] * 3,
        interpret=interpret,
    )(px, py, pz)
    return jnp.stack(out, axis=-1)


def _square_distance(src, dst):
    return (jnp.sum(src ** 2, -1)[:, :, None] + jnp.sum(dst ** 2, -1)[:, None, :]
            - 2.0 * jnp.einsum('bnc,bmc->bnm', src, dst))


def _index_points(points, idx):
    B = points.shape[0]
    bidx = jnp.arange(B).reshape((B,) + (1,) * (idx.ndim - 1))
    return points[bidx, idx]


def _query_ball_point(radius, nsample, xyz, new_xyz):
    B, N, _ = xyz.shape
    sqrdists = _square_distance(new_xyz, xyz)
    gidx = jnp.broadcast_to(jnp.arange(N, dtype=jnp.int32), sqrdists.shape)
    gidx = jnp.where(sqrdists > radius ** 2, N, gidx)
    gidx = jnp.sort(gidx, axis=-1)[:, :, :nsample]
    first = jnp.broadcast_to(gidx[:, :, :1], gidx.shape)
    gidx = jnp.where(gidx == N, first, gidx)
    return gidx


def _shared_mlp(x, layers):
    for W, gamma, beta in layers:
        x = jnp.einsum('...c,oc->...o', x, W)
        mean = jnp.mean(x, axis=(0, 1, 2), keepdims=True)
        var = jnp.var(x, axis=(0, 1, 2), keepdims=True)
        x = (x - mean) / jnp.sqrt(var + 1e-5)
        x = x * gamma + beta
        x = jax.nn.relu(x)
    return x


def _sa_module(xyz, features, npoint, radius, nsample, layers, interpret=False):
    new_xyz = _fps_new_xyz(xyz, npoint, interpret=interpret)
    idx = _query_ball_point(radius, nsample, xyz, new_xyz)
    grouped_xyz = _index_points(xyz, idx) - new_xyz[:, :, None, :]
    if features is not None:
        grouped = jnp.concatenate([grouped_xyz, _index_points(features, idx)], axis=-1)
    else:
        grouped = grouped_xyz
    new_features = jnp.max(_shared_mlp(grouped, layers), axis=2)
    return new_xyz, new_features


def _sa_module_all(xyz, features, layers):
    grouped = xyz[:, None]
    if features is not None:
        grouped = jnp.concatenate([grouped, features[:, None]], axis=-1)
    new_features = jnp.max(_shared_mlp(grouped, layers), axis=2)
    return None, new_features[:, 0]


def kernel(pointcloud, params, interpret=False):
    xyz = pointcloud[..., 0:3]
    features = pointcloud[..., 3:] if pointcloud.shape[-1] > 3 else None
    xyz, features = _sa_module(xyz, features, 512, 0.2, 64, params[0],
                               interpret=interpret)
    xyz, features = _sa_module(xyz, features, 128, 0.4, 64, params[1],
                               interpret=interpret)
    _, features = _sa_module_all(xyz, features, params[2])
    return features


# speedup vs baseline: 2.0818x; 1.7152x over previous
import functools

import jax
import jax.numpy as jnp
from jax.experimental import pallas as pl


def _fps_body(px_ref, py_ref, pz_ref, sx_ref, sy_ref, sz_ref, *, S, N):
    px = px_ref[...]
    py = py_ref[...]
    pz = pz_ref[...]
    B = px.shape[0]
    iota = jax.lax.broadcasted_iota(jnp.int32, (B, N), 1)
    cols = jax.lax.broadcasted_iota(jnp.int32, (B, S), 1)

    def body(s, carry):
        dist, far = carry
        onehot = iota == far
        cx = jnp.sum(jnp.where(onehot, px, 0.0), axis=1, keepdims=True)
        cy = jnp.sum(jnp.where(onehot, py, 0.0), axis=1, keepdims=True)
        cz = jnp.sum(jnp.where(onehot, pz, 0.0), axis=1, keepdims=True)
        hit = cols == s
        sx_ref[...] = jnp.where(hit, cx, sx_ref[...])
        sy_ref[...] = jnp.where(hit, cy, sy_ref[...])
        sz_ref[...] = jnp.where(hit, cz, sz_ref[...])
        dx = px - cx
        dy = py - cy
        dz = pz - cz
        d = (dx * dx + dy * dy) + dz * dz
        dist = jnp.minimum(dist, d)
        dmax = jnp.max(dist, axis=1, keepdims=True)
        far = jnp.min(jnp.where(dist == dmax, iota, N), axis=1, keepdims=True)
        return dist, far

    init = (jnp.full((B, N), 1e10, dtype=jnp.float32),
            jnp.zeros((B, 1), dtype=jnp.int32))
    jax.lax.fori_loop(0, S, body, init)


def _fps_new_xyz(xyz, S, interpret=False):
    B, N, _ = xyz.shape
    px, py, pz = xyz[..., 0], xyz[..., 1], xyz[..., 2]
    out = pl.pallas_call(
        functools.partial(_fps_body, S=S, N=N),
        out_shape=[jax.ShapeDtypeStruct((B, S), jnp.float32)] * 3,
        interpret=interpret,
    )(px, py, pz)
    return jnp.stack(out, axis=-1)


def _mm(x, w):
    return jax.lax.dot_general(x, w, (((1,), (0,)), ((), ())),
                               preferred_element_type=jnp.float32)


def _norm_relu(z, st, gamma, beta, count):
    mu = st[0:1, :] / count
    var = st[1:2, :] / count - mu * mu
    inv = jax.lax.rsqrt(var + 1e-5)
    return jnp.maximum(z * (inv * gamma)[None] + (beta - mu[0] * inv[0] * gamma)[None, None, :], 0.0)


def _first_body(g_ref, c_ref, w_ref, z_ref, st_ref):
    g = g_ref[...]
    x = g - c_ref[...][:, None, :]
    RB, K, Cp = x.shape
    z = _mm(x.reshape(RB * K, Cp), w_ref[...])
    z_ref[...] = z.reshape(RB, K, -1)
    @pl.when(pl.program_id(0) == 0)
    def _():
        st_ref[...] = jnp.zeros_like(st_ref)
    st_ref[...] += jnp.concatenate(
        [jnp.sum(z, 0, keepdims=True), jnp.sum(z * z, 0, keepdims=True)], 0)


def _mid_body(z_ref, st_ref, gb_ref, w_ref, zo_ref, sto_ref, *, count):
    z = z_ref[...]
    RB, K, C = z.shape
    gb = gb_ref[...]
    h = _norm_relu(z, st_ref[...], gb[0], gb[1], count)
    zo = _mm(h.reshape(RB * K, C), w_ref[...])
    zo_ref[...] = zo.reshape(RB, K, -1)
    @pl.when(pl.program_id(0) == 0)
    def _():
        sto_ref[...] = jnp.zeros_like(sto_ref)
    sto_ref[...] += jnp.concatenate(
        [jnp.sum(zo, 0, keepdims=True), jnp.sum(zo * zo, 0, keepdims=True)], 0)


def _last_body(z_ref, st_ref, gb_ref, w_ref, zm_ref, sto_ref, *, count):
    z = z_ref[...]
    RB, K, C = z.shape
    gb = gb_ref[...]
    h = _norm_relu(z, st_ref[...], gb[0], gb[1], count)
    zo = _mm(h.reshape(RB * K, C), w_ref[...])
    @pl.when(pl.program_id(0) == 0)
    def _():
        sto_ref[...] = jnp.zeros_like(sto_ref)
    sto_ref[...] += jnp.concatenate(
        [jnp.sum(zo, 0, keepdims=True), jnp.sum(zo * zo, 0, keepdims=True)], 0)
    zm_ref[...] = jnp.max(zo.reshape(RB, K, -1), axis=1)


def _final_body(zm_ref, st_ref, gb_ref, f_ref, *, count):
    gb = gb_ref[...]
    st = st_ref[...]
    mu = st[0:1, :] / count
    var = st[1:2, :] / count - mu * mu
    inv = jax.lax.rsqrt(var + 1e-5)
    f_ref[...] = jnp.maximum((zm_ref[...] - mu) * (inv * gb[0:1]) + gb[1:2], 0.0)


def _mlp_maxpool(G, C, layers, rb, interpret=False):
    R, K, Cp = G.shape
    count = float(R * K)
    grid = R // rb
    w1, g1, b1 = layers[0]
    C1 = w1.shape[1]

    z, st = pl.pallas_call(
        _first_body,
        grid=(grid,),
        interpret=interpret,
        in_specs=[pl.BlockSpec((rb, K, Cp), lambda i: (i, 0, 0)),
                  pl.BlockSpec((rb, Cp), lambda i: (i, 0)),
                  pl.BlockSpec((Cp, C1), lambda i: (0, 0))],
        out_specs=[pl.BlockSpec((rb, K, C1), lambda i: (i, 0, 0)),
                   pl.BlockSpec((2, C1), lambda i: (0, 0))],
        out_shape=[jax.ShapeDtypeStruct((R, K, C1), jnp.float32),
                   jax.ShapeDtypeStruct((2, C1), jnp.float32)],
    )(G, C, w1)

    for li in range(1, len(layers)):
        wT = layers[li][0]
        Cin, Cout = wT.shape
        gb = jnp.stack([layers[li - 1][1], layers[li - 1][2]], 0)
        body = _last_body if li == len(layers) - 1 else _mid_body
        outs = [jax.ShapeDtypeStruct(
                    (R, Cout) if li == len(layers) - 1 else (R, K, Cout),
                    jnp.float32),
                jax.ShapeDtypeStruct((2, Cout), jnp.float32)]
        out_specs = [pl.BlockSpec((rb, Cout), lambda i: (i, 0))
                     if li == len(layers) - 1 else
                     pl.BlockSpec((rb, K, Cout), lambda i: (i, 0, 0)),
                     pl.BlockSpec((2, Cout), lambda i: (0, 0))]
        z, st = pl.pallas_call(
            functools.partial(body, count=count),
            grid=(grid,),
            interpret=interpret,
            in_specs=[pl.BlockSpec((rb, K, Cin), lambda i: (i, 0, 0)),
                      pl.BlockSpec((2, Cin), lambda i: (0, 0)),
                      pl.BlockSpec((2, Cin), lambda i: (0, 0)),
                      pl.BlockSpec((Cin, Cout), lambda i: (0, 0))],
            out_specs=out_specs,
            out_shape=outs,
        )(z, st, gb, wT)

    wT, gm, bt = layers[-1]
    gb = jnp.stack([gm, bt], 0)
    Cl = wT.shape[1]
    f = pl.pallas_call(
        functools.partial(_final_body, count=count),
        interpret=interpret,
        in_specs=[pl.BlockSpec((R, Cl), lambda: (0, 0)),
                  pl.BlockSpec((2, Cl), lambda: (0, 0)),
                  pl.BlockSpec((2, Cl), lambda: (0, 0))],
        out_specs=pl.BlockSpec((R, Cl), lambda: (0, 0)),
        out_shape=jax.ShapeDtypeStruct((R, Cl), jnp.float32),
    )(z, st, gb)
    return f


def _norm_relu2(z, gamma, beta, count):
    s = jnp.sum(z, 0, keepdims=True)
    q = jnp.sum(z * z, 0, keepdims=True)
    mu = s / count
    var = q / count - mu * mu
    inv = jax.lax.rsqrt(var + 1e-5)
    return jnp.maximum((z - mu) * (inv * gamma[None]) + beta[None], 0.0)


def _sa3_body(r_ref, w1_ref, w2_ref, w3_ref, gb1_ref, gb2_ref, gb3_ref,
              o_ref, *, Bb):
    x = r_ref[...]
    R3 = x.shape[0]
    count = float(R3)
    h1 = _norm_relu2(_mm(x, w1_ref[...]), gb1_ref[0], gb1_ref[1], count)
    h2 = _norm_relu2(_mm(h1, w2_ref[...]), gb2_ref[0], gb2_ref[1], count)
    z3 = _mm(h2, w3_ref[...])
    s = jnp.sum(z3, 0, keepdims=True)
    q = jnp.sum(z3 * z3, 0, keepdims=True)
    mu = s / count
    var = q / count - mu * mu
    inv = jax.lax.rsqrt(var + 1e-5)
    zm = jnp.max(z3.reshape(Bb, R3 // Bb, -1), 1)
    o_ref[...] = jnp.maximum((zm - mu) * (inv * gb3_ref[0][None])
                             + gb3_ref[1][None], 0.0)


def _sa3(rows, layers, Bb, interpret=False):
    (w1, g1, b1), (w2, g2, b2), (w3, g3, b3) = layers
    Cl = w3.shape[1]
    return pl.pallas_call(
        functools.partial(_sa3_body, Bb=Bb),
        interpret=interpret,
        out_shape=jax.ShapeDtypeStruct((Bb, Cl), jnp.float32),
    )(rows, w1, w2, w3,
      jnp.stack([g1, b1], 0), jnp.stack([g2, b2], 0), jnp.stack([g3, b3], 0))


def _pad_layers(layers, cpad):
    out = []
    for i, (W, gm, bt) in enumerate(layers):
        wT = W.T
        if i == 0:
            wT = jnp.pad(wT, ((0, cpad - wT.shape[0]), (0, 0)))
        out.append((wT, gm, bt))
    return out


def _pad_rows(a, cpad):
    c = a.shape[-1]
    a2 = a.reshape(-1, c)
    return jnp.pad(a2, ((0, 0), (0, cpad - c)))


def _square_distance(src, dst):
    return (jnp.sum(src ** 2, -1)[:, :, None] + jnp.sum(dst ** 2, -1)[:, None, :]
            - 2.0 * jnp.einsum('bnc,bmc->bnm', src, dst))


def _index_points(points, idx):
    B = points.shape[0]
    bidx = jnp.arange(B).reshape((B,) + (1,) * (idx.ndim - 1))
    return points[bidx, idx]


def _query_ball_point(radius, nsample, xyz, new_xyz):
    B, N, _ = xyz.shape
    sqrdists = _square_distance(new_xyz, xyz)
    gidx = jnp.broadcast_to(jnp.arange(N, dtype=jnp.int32), sqrdists.shape)
    gidx = jnp.where(sqrdists > radius ** 2, N, gidx)
    gidx = jnp.sort(gidx, axis=-1)[:, :, :nsample]
    first = jnp.broadcast_to(gidx[:, :, :1], gidx.shape)
    gidx = jnp.where(gidx == N, first, gidx)
    return gidx


def _gather_rows_xla(src, idx):
    B, S, K = idx.shape
    return _index_points(src, idx).reshape(B * S, K, src.shape[-1])


def kernel(pointcloud, params, interpret=False):
    B, N, _ = pointcloud.shape
    xyz = pointcloud[..., 0:3]

    S1, K1, CP1 = 512, 64, 16
    new1 = _fps_new_xyz(xyz, S1, interpret=interpret)
    idx1 = _query_ball_point(0.2, K1, xyz, new1)
    src1 = _pad_rows(pointcloud, CP1).reshape(B, N, CP1)
    G1 = _gather_rows_xla(src1, idx1)
    C1 = _pad_rows(new1, CP1)
    f1 = _mlp_maxpool(G1, C1, _pad_layers(params[0], CP1), rb=64, interpret=interpret)

    S2, K2, CP2 = 128, 64, 144
    new2 = _fps_new_xyz(new1, S2, interpret=interpret)
    idx2 = _query_ball_point(0.4, K2, new1, new2)
    src2 = jnp.concatenate(
        [new1.reshape(B * S1, 3), f1,
         jnp.zeros((B * S1, CP2 - 3 - f1.shape[1]), jnp.float32)], 1)
    G2 = _gather_rows_xla(src2.reshape(B, S1, CP2), idx2)
    C2 = _pad_rows(new2, CP2)
    f2 = _mlp_maxpool(G2, C2, _pad_layers(params[1], CP2), rb=32, interpret=interpret)

    CP3 = 272
    rows3 = jnp.concatenate(
        [new2.reshape(B * S2, 3), f2,
         jnp.zeros((B * S2, CP3 - 3 - f2.shape[1]), jnp.float32)], 1)
    return _sa3(rows3, _pad_layers(params[2], CP3), B, interpret=interpret)
